# initial kernel scaffold (unmeasured)
import jax
import jax.numpy as jnp
from jax import lax
from jax.experimental import pallas as pl
from jax.experimental.pallas import tpu as pltpu

M_PER = 4096
N = 4096
K_PER = 2048

ROW_CHUNK = 512
N_CHUNKS = M_PER // ROW_CHUNK


def _allreduce_y_body(p_ref, o_ref, recv_ref, loc_ref, sum_ref,
                      copy_sem, out_sem, send_sem, recv_sem):
    my_x = lax.axis_index("x")
    my_y = lax.axis_index("y")

    rdma = pltpu.make_async_remote_copy(
        src_ref=p_ref,
        dst_ref=recv_ref,
        send_sem=send_sem,
        recv_sem=recv_sem,
        device_id=(my_x, 1 - my_y),
        device_id_type=pl.DeviceIdType.MESH,
    )
    rdma.start()
    rdma.wait()

    for h in range(N_CHUNKS):
        sl = pl.ds(h * ROW_CHUNK, ROW_CHUNK)
        cp = pltpu.make_async_copy(p_ref.at[sl], loc_ref, copy_sem)
        cp.start()
        cp.wait()
        sum_ref[...] = loc_ref[...] + recv_ref[sl]
        ocp = pltpu.make_async_copy(sum_ref, o_ref.at[sl], out_sem)
        ocp.start()
        ocp.wait()


def kernel(A, B):
    a16 = A.astype(jnp.bfloat16)
    b16 = B.astype(jnp.bfloat16)
    partial = jnp.dot(a16, b16, preferred_element_type=jnp.bfloat16)

    return pl.pallas_call(
        _allreduce_y_body,
        out_shape=jax.ShapeDtypeStruct((M_PER, N), jnp.bfloat16),
        in_specs=[pl.BlockSpec(memory_space=pl.ANY)],
        out_specs=pl.BlockSpec(memory_space=pl.ANY),
        scratch_shapes=[
            pltpu.VMEM((M_PER, N), jnp.bfloat16),
            pltpu.VMEM((ROW_CHUNK, N), jnp.bfloat16),
            pltpu.VMEM((ROW_CHUNK, N), jnp.bfloat16),
            pltpu.SemaphoreType.DMA,
            pltpu.SemaphoreType.DMA,
            pltpu.SemaphoreType.DMA,
            pltpu.SemaphoreType.DMA,
        ],
        compiler_params=pltpu.CompilerParams(collective_id=0),
    )(partial)


# baseline (device time: 530829 ns/iter reference)
import jax
import jax.numpy as jnp
from jax import lax
from jax.experimental import pallas as pl
from jax.experimental.pallas import tpu as pltpu

M_PER = 4096
N = 4096
K_PER = 2048

ROW_CHUNK = 512
N_CHUNKS = M_PER // ROW_CHUNK


def _allreduce_y_body(p_ref, o_ref, recv_ref, loc_ref, sum_ref,
                      copy_sem, out_sem, send_sem, recv_sem):
    my_x = lax.axis_index("x")
    my_y = lax.axis_index("y")

    rdma = pltpu.make_async_remote_copy(
        src_ref=p_ref,
        dst_ref=recv_ref,
        send_sem=send_sem,
        recv_sem=recv_sem,
        device_id=(my_x, 1 - my_y),
        device_id_type=pl.DeviceIdType.MESH,
    )
    rdma.start()
    rdma.wait()

    for h in range(N_CHUNKS):
        sl = pl.ds(h * ROW_CHUNK, ROW_CHUNK)
        cp = pltpu.make_async_copy(p_ref.at[sl], loc_ref, copy_sem)
        cp.start()
        cp.wait()
        sum_ref[...] = loc_ref[...] + recv_ref[sl]
        ocp = pltpu.make_async_copy(sum_ref, o_ref.at[sl], out_sem)
        ocp.start()
        ocp.wait()


def kernel(A, B):
    a16 = A.astype(jnp.bfloat16)
    b16 = B.astype(jnp.bfloat16)
    partial = jnp.dot(a16, b16, preferred_element_type=jnp.bfloat16)

    return pl.pallas_call(
        _allreduce_y_body,
        out_shape=jax.ShapeDtypeStruct((M_PER, N), jnp.bfloat16),
        in_specs=[pl.BlockSpec(memory_space=pl.ANY)],
        out_specs=pl.BlockSpec(memory_space=pl.ANY),
        scratch_shapes=[
            pltpu.VMEM((M_PER, N), jnp.bfloat16),
            pltpu.VMEM((ROW_CHUNK, N), jnp.bfloat16),
            pltpu.VMEM((ROW_CHUNK, N), jnp.bfloat16),
            pltpu.SemaphoreType.DMA,
            pltpu.SemaphoreType.DMA,
            pltpu.SemaphoreType.DMA,
            pltpu.SemaphoreType.DMA,
        ],
        compiler_params=pltpu.CompilerParams(
            vmem_limit_bytes=56 * 1024 * 1024,
        ),
    )(partial)


# device time: 444689 ns/iter; 1.1937x vs baseline; 1.1937x over previous
import jax
import jax.numpy as jnp
from jax import lax
from jax.experimental import pallas as pl
from jax.experimental.pallas import tpu as pltpu

M_PER = 4096
N = 4096
K_PER = 2048

ROW_CHUNK = 512
N_CHUNKS = M_PER // ROW_CHUNK


def _fused_body(a_ref, b_ref, o_ref, send_buf, recv_buf,
                send_sems, recv_sems, out_sems, credit_sem):
    my_x = lax.axis_index("x")
    my_y = lax.axis_index("y")
    nbr = (my_x, 1 - my_y)

    rdmas = [None] * N_CHUNKS
    out_dmas = [None] * N_CHUNKS

    def process_chunk(c):
        s = c % 2
        rdmas[c].wait_recv()
        recv_buf[s] = recv_buf[s] + send_buf[s]
        out_dmas[c] = pltpu.make_async_copy(
            recv_buf.at[s],
            o_ref.at[pl.ds(c * ROW_CHUNK, ROW_CHUNK)],
            out_sems.at[s],
        )
        out_dmas[c].start()
        out_dmas[c].wait()
        if c < N_CHUNKS - 2:
            pl.semaphore_signal(
                credit_sem, inc=1,
                device_id=nbr, device_id_type=pl.DeviceIdType.MESH,
            )

    for j in range(N_CHUNKS):
        s = j % 2
        if j >= 2:
            rdmas[j - 2].wait_send()
        sl = pl.ds(j * ROW_CHUNK, ROW_CHUNK)
        send_buf[s] = jnp.dot(
            a_ref[sl], b_ref[...], preferred_element_type=jnp.float32
        ).astype(jnp.bfloat16)
        if j >= 2:
            pl.semaphore_wait(credit_sem, 1)
        rdmas[j] = pltpu.make_async_remote_copy(
            src_ref=send_buf.at[s],
            dst_ref=recv_buf.at[s],
            send_sem=send_sems.at[s],
            recv_sem=recv_sems.at[s],
            device_id=nbr,
            device_id_type=pl.DeviceIdType.MESH,
        )
        rdmas[j].start()
        if j >= 1:
            process_chunk(j - 1)

    process_chunk(N_CHUNKS - 1)
    rdmas[N_CHUNKS - 2].wait_send()
    rdmas[N_CHUNKS - 1].wait_send()


def kernel(A, B):
    a16 = A.astype(jnp.bfloat16)
    b16 = B.astype(jnp.bfloat16)

    return pl.pallas_call(
        _fused_body,
        out_shape=jax.ShapeDtypeStruct((M_PER, N), jnp.bfloat16),
        in_specs=[
            pl.BlockSpec(memory_space=pltpu.MemorySpace.VMEM),
            pl.BlockSpec(memory_space=pltpu.MemorySpace.VMEM),
        ],
        out_specs=pl.BlockSpec(memory_space=pl.ANY),
        scratch_shapes=[
            pltpu.VMEM((2, ROW_CHUNK, N), jnp.bfloat16),
            pltpu.VMEM((2, ROW_CHUNK, N), jnp.bfloat16),
            pltpu.SemaphoreType.DMA((2,)),
            pltpu.SemaphoreType.DMA((2,)),
            pltpu.SemaphoreType.DMA((2,)),
            pltpu.SemaphoreType.REGULAR,
        ],
        compiler_params=pltpu.CompilerParams(
            vmem_limit_bytes=60 * 1024 * 1024,
        ),
    )(a16, b16)


# device time: 350534 ns/iter; 1.5143x vs baseline; 1.2686x over previous
import jax
import jax.numpy as jnp
from jax import lax
from jax.experimental import pallas as pl
from jax.experimental.pallas import tpu as pltpu

M_PER = 4096
N = 4096
K_PER = 2048

RCH = 512
NR = M_PER // RCH
HALF_K = K_PER // 2
BCH = 256
NB = HALF_K // BCH

_MESH = pl.DeviceIdType.MESH


def _v3_body(a_ref, b_ref, o_ref, a_nbr_hbm, b_nbr, aslots, stage,
             ysend, fsend, by_recv, bx_recv, a_recv,
             cp_sems, rb_sems, ot_sems):
    my_x = lax.axis_index("x")
    my_y = lax.axis_index("y")
    nbr_y = (my_x, 1 - my_y)
    nbr_x = (1 - my_x, my_y)

    def b_rows_mine(c):
        return pl.ds(my_x * HALF_K + c * BCH, BCH)

    def b_rows_other(c):
        return pl.ds((1 - my_x) * HALF_K + c * BCH, BCH)

    def b_y_rdma(c):
        return pltpu.make_async_remote_copy(
            src_ref=b_ref.at[b_rows_mine(c)],
            dst_ref=b_nbr.at[b_rows_mine(c)],
            send_sem=ysend.at[c],
            recv_sem=by_recv.at[c],
            device_id=nbr_y,
            device_id_type=_MESH,
        )

    def a_y_rdma(c):
        rows = pl.ds(c * RCH, RCH)
        return pltpu.make_async_remote_copy(
            src_ref=a_ref.at[rows],
            dst_ref=a_nbr_hbm.at[rows],
            send_sem=ysend.at[NB + c],
            recv_sem=a_recv.at[c],
            device_id=nbr_y,
            device_id_type=_MESH,
        )

    def b_x_rdma(c):
        return pltpu.make_async_remote_copy(
            src_ref=b_nbr.at[b_rows_mine(c)],
            dst_ref=b_nbr.at[b_rows_mine(c)],
            send_sem=fsend.at[c],
            recv_sem=bx_recv.at[c],
            device_id=nbr_x,
            device_id_type=_MESH,
        )

    def b_x_wait_rdma(c):
        return pltpu.make_async_remote_copy(
            src_ref=b_nbr.at[b_rows_other(c)],
            dst_ref=b_nbr.at[b_rows_other(c)],
            send_sem=fsend.at[c],
            recv_sem=bx_recv.at[c],
            device_id=nbr_x,
            device_id_type=_MESH,
        )

    def _send_b(c, x):
        b_y_rdma(c).start()
        return x

    lax.fori_loop(0, NB, _send_b, 0)

    def _send_a(c, x):
        a_y_rdma(c).start()
        return x

    lax.fori_loop(0, NR, _send_a, 0)

    def _phase1(r, x):
        s = r % 2
        rows = pl.ds(r * RCH, RCH)
        cp = pltpu.make_async_copy(a_ref.at[rows], aslots.at[s], cp_sems.at[s])
        cp.start()

        @pl.when(r < NB)
        def _():
            b_y_rdma(r).wait_recv()
            b_x_rdma(r).start()

        cp.wait()
        stage[s] = jnp.dot(
            aslots[s], b_ref[...], preferred_element_type=jnp.float32
        ).astype(jnp.bfloat16)
        od = pltpu.make_async_copy(stage.at[s], o_ref.at[rows], ot_sems.at[s])
        od.start()
        od.wait()
        return x

    lax.fori_loop(0, NR, _phase1, 0)

    def _phase2(c, x):
        b_x_wait_rdma(c).wait_recv()
        return x

    lax.fori_loop(0, NB, _phase2, 0)

    def _phase3(r, x):
        s = r % 2
        rows = pl.ds(r * RCH, RCH)
        a_y_rdma(r).wait_recv()
        ca = pltpu.make_async_copy(
            a_nbr_hbm.at[rows], aslots.at[s], cp_sems.at[s])
        ca.start()
        rb = pltpu.make_async_copy(o_ref.at[rows], stage.at[s], rb_sems.at[s])
        rb.start()
        ca.wait()
        p2 = jnp.dot(
            aslots[s], b_nbr[...], preferred_element_type=jnp.float32
        )
        rb.wait()
        stage[s] = stage[s] + p2.astype(jnp.bfloat16)
        od = pltpu.make_async_copy(stage.at[s], o_ref.at[rows], ot_sems.at[s])
        od.start()
        od.wait()
        return x

    lax.fori_loop(0, NR, _phase3, 0)

    def _drain_b(c, x):
        b_y_rdma(c).wait_send()
        return x

    lax.fori_loop(0, NB, _drain_b, 0)

    def _drain_a(c, x):
        a_y_rdma(c).wait_send()
        return x

    lax.fori_loop(0, NR, _drain_a, 0)

    def _drain_f(c, x):
        b_x_rdma(c).wait_send()
        return x

    lax.fori_loop(0, NB, _drain_f, 0)


def kernel(A, B):
    a16 = A.astype(jnp.bfloat16)
    b16 = B.astype(jnp.bfloat16)

    out, _ = pl.pallas_call(
        _v3_body,
        out_shape=[
            jax.ShapeDtypeStruct((M_PER, N), jnp.bfloat16),
            jax.ShapeDtypeStruct((M_PER, K_PER), jnp.bfloat16),
        ],
        in_specs=[
            pl.BlockSpec(memory_space=pl.ANY),
            pl.BlockSpec(memory_space=pltpu.MemorySpace.VMEM),
        ],
        out_specs=[
            pl.BlockSpec(memory_space=pl.ANY),
            pl.BlockSpec(memory_space=pl.ANY),
        ],
        scratch_shapes=[
            pltpu.VMEM((K_PER, N), jnp.bfloat16),
            pltpu.VMEM((2, RCH, K_PER), jnp.bfloat16),
            pltpu.VMEM((2, RCH, N), jnp.bfloat16),
            pltpu.SemaphoreType.DMA((NB + NR,)),
            pltpu.SemaphoreType.DMA((NB,)),
            pltpu.SemaphoreType.DMA((NB,)),
            pltpu.SemaphoreType.DMA((NB,)),
            pltpu.SemaphoreType.DMA((NR,)),
            pltpu.SemaphoreType.DMA((2,)),
            pltpu.SemaphoreType.DMA((2,)),
            pltpu.SemaphoreType.DMA((2,)),
        ],
        compiler_params=pltpu.CompilerParams(
            vmem_limit_bytes=60 * 1024 * 1024,
        ),
    )(a16, b16)
    return out
